# Initial kernel scaffold; baseline (speedup 1.0000x reference)
import functools

import jax
import jax.numpy as jnp
from jax import lax
from jax.experimental import pallas as pl
from jax.experimental.pallas import tpu as pltpu
from jax.experimental.pallas import tpu_sc as plsc

N = 10000
E = 160000
D_IN = 256
D_H = 256
D_OUT = 128
HALF = 128

NC = 2
NS = 16
CH = 80
EP = 163840
RPC = EP // CH
NPAD = 16
NACC = N + NPAD
ZROWS = NACC // NS
OROWS = N // NS

_MESH = plsc.VectorSubcoreMesh(core_axis_name="c", subcore_axis_name="s")


@functools.partial(
    pl.kernel,
    mesh=_MESH,
    out_type=jax.ShapeDtypeStruct((4, N, 16), jnp.float32),
    scratch_types=[
        pltpu.VMEM((64, CH), jnp.int32),
        pltpu.VMEM((64, CH), jnp.int32),
        pltpu.VMEM((CH, 16), jnp.float32),
        pltpu.VMEM_SHARED((NACC, 16), jnp.float32),
        pltpu.VMEM_SHARED((NACC, 16), jnp.float32),
    ],
)
def _degrees(src_hbm, dst_hbm, ones_hbm, z16_hbm, out_hbm,
             sidx, didx, ones_v, acc_o, acc_i):
    c = lax.axis_index("c")
    s = lax.axis_index("s")
    w = s * NC + c

    pltpu.sync_copy(z16_hbm, acc_o.at[pl.ds(s * ZROWS, ZROWS)])
    pltpu.sync_copy(z16_hbm, acc_i.at[pl.ds(s * ZROWS, ZROWS)])
    pltpu.sync_copy(src_hbm.at[pl.ds(w * 64, 64)], sidx)
    pltpu.sync_copy(dst_hbm.at[pl.ds(w * 64, 64)], didx)
    pltpu.sync_copy(ones_hbm, ones_v)
    plsc.subcore_barrier()

    def body(j, carry):
        pltpu.sync_copy(ones_v, acc_o.at[sidx.at[j]], add=True)
        pltpu.sync_copy(ones_v, acc_i.at[didx.at[j]], add=True)
        return carry

    lax.fori_loop(0, 64, body, 0, unroll=False)
    plsc.subcore_barrier()

    r0 = s * OROWS
    pltpu.sync_copy(acc_o.at[pl.ds(r0, OROWS)],
                    out_hbm.at[c * 2].at[pl.ds(r0, OROWS)])
    pltpu.sync_copy(acc_i.at[pl.ds(r0, OROWS)],
                    out_hbm.at[c * 2 + 1].at[pl.ds(r0, OROWS)])


def _make_agg(col_split):
    rows_per_tile = RPC // NS if col_split else RPC // (NC * NS)

    @functools.partial(
        pl.kernel,
        mesh=_MESH,
        out_type=jax.ShapeDtypeStruct((2, N, HALF), jnp.float32),
        scratch_types=[
            pltpu.VMEM((rows_per_tile, CH), jnp.int32),
            pltpu.VMEM((rows_per_tile, CH), jnp.int32),
            pltpu.VMEM((CH, HALF), jnp.float32),
            pltpu.VMEM_SHARED((NACC, HALF), jnp.float32),
            pltpu.SemaphoreType.DMA,
        ],
    )
    def agg(src_hbm, dst_hbm, tbl_hbm, z_hbm, out_hbm,
            sidx, didx, rowbuf, acc, sem):
        c = lax.axis_index("c")
        s = lax.axis_index("s")

        pltpu.sync_copy(z_hbm, acc.at[pl.ds(s * ZROWS, ZROWS)])
        if col_split:
            base = s * rows_per_tile
        else:
            base = c * (RPC // NC) + s * rows_per_tile
        pltpu.sync_copy(src_hbm.at[pl.ds(base, rows_per_tile)], sidx)
        pltpu.sync_copy(dst_hbm.at[pl.ds(base, rows_per_tile)], didx)
        plsc.subcore_barrier()

        def body(j, carry):
            if col_split:
                tbl = tbl_hbm.at[c]
            else:
                tbl = tbl_hbm
            pltpu.async_copy(tbl.at[sidx.at[j]], rowbuf, sem).wait()
            pltpu.sync_copy(rowbuf, acc.at[didx.at[j]], add=True)
            return carry

        lax.fori_loop(0, rows_per_tile, body, 0, unroll=False)
        plsc.subcore_barrier()

        r0 = s * OROWS
        pltpu.sync_copy(acc.at[pl.ds(r0, OROWS)],
                        out_hbm.at[c].at[pl.ds(r0, OROWS)])

    return agg


_agg_l1 = _make_agg(col_split=True)
_agg_l2 = _make_agg(col_split=False)


_NB = 1000


def _norms(degs_blk):
    outd = degs_blk[0, :, 0:1] + degs_blk[2, :, 0:1]
    ind = degs_blk[1, :, 0:1] + degs_blk[3, :, 0:1]
    ns = jnp.where(outd > 0, lax.rsqrt(jnp.maximum(outd, 1e-12)), 0.0)
    nd = jnp.where(ind > 0, lax.rsqrt(jnp.maximum(ind, 1e-12)), 0.0)
    return ns, nd


def _tc_pre_body(x_ref, w1_ref, degs_ref, out_ref):
    ns, _ = _norms(degs_ref[...])
    y = jnp.dot(x_ref[...], w1_ref[...], preferred_element_type=jnp.float32)
    y = y * ns
    out_ref[0] = y[:, :HALF]
    out_ref[1] = y[:, HALF:]


def _tc_mid_body(agg_ref, degs_ref, b1_ref, w2_ref, out_ref):
    ns, nd = _norms(degs_ref[...])
    h = jnp.concatenate([agg_ref[0], agg_ref[1]], axis=-1)
    h = jnp.maximum(h * nd + b1_ref[...], 0.0)
    y = jnp.dot(h, w2_ref[...], preferred_element_type=jnp.float32)
    out_ref[...] = y * ns


def _tc_post_body(parts_ref, degs_ref, b2_ref, out_ref):
    _, nd = _norms(degs_ref[...])
    out_ref[...] = (parts_ref[0] + parts_ref[1]) * nd + b2_ref[...]


def _tc_pre(x, w1, degs):
    return pl.pallas_call(
        _tc_pre_body,
        grid=(N // _NB,),
        in_specs=[
            pl.BlockSpec((_NB, D_IN), lambda i: (i, 0)),
            pl.BlockSpec((D_IN, D_H), lambda i: (0, 0)),
            pl.BlockSpec((4, _NB, 16), lambda i: (0, i, 0)),
        ],
        out_specs=pl.BlockSpec((2, _NB, HALF), lambda i: (0, i, 0)),
        out_shape=jax.ShapeDtypeStruct((2, N, HALF), jnp.float32),
    )(x, w1, degs)


def _tc_mid(agg1, degs, b1, w2):
    return pl.pallas_call(
        _tc_mid_body,
        grid=(N // _NB,),
        in_specs=[
            pl.BlockSpec((2, _NB, HALF), lambda i: (0, i, 0)),
            pl.BlockSpec((4, _NB, 16), lambda i: (0, i, 0)),
            pl.BlockSpec((1, D_H), lambda i: (0, 0)),
            pl.BlockSpec((D_H, D_OUT), lambda i: (0, 0)),
        ],
        out_specs=pl.BlockSpec((_NB, D_OUT), lambda i: (i, 0)),
        out_shape=jax.ShapeDtypeStruct((N, D_OUT), jnp.float32),
    )(agg1, degs, b1, w2)


def _tc_post(parts, degs, b2):
    return pl.pallas_call(
        _tc_post_body,
        grid=(N // _NB,),
        in_specs=[
            pl.BlockSpec((2, _NB, HALF), lambda i: (0, i, 0)),
            pl.BlockSpec((4, _NB, 16), lambda i: (0, i, 0)),
            pl.BlockSpec((1, D_OUT), lambda i: (0, 0)),
        ],
        out_specs=pl.BlockSpec((_NB, D_OUT), lambda i: (i, 0)),
        out_shape=jax.ShapeDtypeStruct((N, D_OUT), jnp.float32),
    )(parts, degs, b2)


def kernel(features, edge_index, W1, b1, W2, b2):
    src = edge_index[0]
    dst = edge_index[1]
    npad = EP - E
    pad_src = jnp.zeros((npad,), jnp.int32)
    pad_dst = N + (jnp.arange(npad, dtype=jnp.int32) % NPAD)
    src_p = jnp.concatenate([src.astype(jnp.int32), pad_src]).reshape(RPC, CH)
    dst_p = jnp.concatenate([dst.astype(jnp.int32), pad_dst]).reshape(RPC, CH)

    ones16 = jnp.ones((CH, 16), jnp.float32)
    z16 = jnp.zeros((ZROWS, 16), jnp.float32)
    zrows = jnp.zeros((ZROWS, HALF), jnp.float32)

    degs = _degrees(src_p, dst_p, ones16, z16)
    hw1 = _tc_pre(features, W1, degs)
    agg1 = _agg_l1(src_p, dst_p, hw1, zrows)
    hw2 = _tc_mid(agg1, degs, b1.reshape(1, D_H), W2)
    parts = _agg_l2(src_p, dst_p, hw2, zrows)
    out = _tc_post(parts, degs, b2.reshape(1, D_OUT))
    return out


# speedup vs baseline: 2.9631x; 2.9631x over previous
import functools

import jax
import jax.numpy as jnp
from jax import lax
from jax.experimental import pallas as pl
from jax.experimental.pallas import tpu as pltpu
from jax.experimental.pallas import tpu_sc as plsc

N = 10000
E = 160000
D_IN = 256
D_H = 256
D_OUT = 128
HALF = 128

NC = 2
NS = 16
CH = 80
EP = 163840
RPC = EP // CH
NPAD = 16
ZROWS = 640
NACC = NS * ZROWS
OR_LO = 624
OR_HI = N - 15 * OR_LO

_MESH = plsc.VectorSubcoreMesh(core_axis_name="c", subcore_axis_name="s",
                               num_cores=NC, num_subcores=NS)


def _epilogue(s, acc, out):
    @pl.when(s < 15)
    def _lo():
        r0 = pl.multiple_of(s * OR_LO, 8)
        pltpu.sync_copy(acc.at[pl.ds(r0, OR_LO)], out.at[pl.ds(r0, OR_LO)])

    @pl.when(s == 15)
    def _hi():
        r0 = 15 * OR_LO
        pltpu.sync_copy(acc.at[pl.ds(r0, OR_HI)], out.at[pl.ds(r0, OR_HI)])


def _degrees_body(src_hbm, dst_hbm, ones_hbm, z_hbm, out_hbm,
                  sidx, didx, ones_v, acc):
    c = lax.axis_index("c")
    s = lax.axis_index("s")
    w = s * NC + c

    z0 = pl.multiple_of(s * ZROWS, 8)
    e0 = pl.multiple_of(w * 64, 8)
    pltpu.sync_copy(src_hbm.at[pl.ds(e0, 64)], sidx)
    pltpu.sync_copy(dst_hbm.at[pl.ds(e0, 64)], didx)
    pltpu.sync_copy(ones_hbm, ones_v)

    for which, idx in ((0, sidx), (1, didx)):
        pltpu.sync_copy(z_hbm, acc.at[pl.ds(z0, ZROWS)])
        plsc.subcore_barrier()

        def body(j, carry, idx=idx):
            pltpu.sync_copy(ones_v, acc.at[idx.at[j]], add=True)
            return carry

        lax.fori_loop(0, 64, body, 0, unroll=False)
        plsc.subcore_barrier()
        _epilogue(s, acc, out_hbm.at[c * 2 + which])
        plsc.subcore_barrier()


_degrees = functools.partial(
    pl.kernel,
    mesh=_MESH,
    out_type=jax.ShapeDtypeStruct((4, N, HALF), jnp.float32),
    scratch_types=[
        pltpu.VMEM((64, CH), jnp.int32),
        pltpu.VMEM((64, CH), jnp.int32),
        pltpu.VMEM((CH, HALF), jnp.float32),
        pltpu.VMEM_SHARED((NACC, HALF), jnp.float32),
    ],
)(_degrees_body)


def _make_agg_body(col_split):
    rows_per_tile = RPC // NS if col_split else RPC // (NC * NS)

    def agg(src_hbm, dst_hbm, tbl_hbm, z_hbm, out_hbm,
            sidx, didx, rowbuf, acc, sem):
        c = lax.axis_index("c")
        s = lax.axis_index("s")

        z0 = pl.multiple_of(s * ZROWS, 8)
        pltpu.sync_copy(z_hbm, acc.at[pl.ds(z0, ZROWS)])
        if col_split:
            base = s * rows_per_tile
        else:
            base = c * (RPC // NC) + s * rows_per_tile
        base = pl.multiple_of(base, 8)
        pltpu.sync_copy(src_hbm.at[pl.ds(base, rows_per_tile)], sidx)
        pltpu.sync_copy(dst_hbm.at[pl.ds(base, rows_per_tile)], didx)
        plsc.subcore_barrier()

        def body(j, carry):
            if col_split:
                tbl = tbl_hbm.at[c]
            else:
                tbl = tbl_hbm
            pltpu.async_copy(tbl.at[sidx.at[j]], rowbuf, sem).wait()
            pltpu.sync_copy(rowbuf, acc.at[didx.at[j]], add=True)
            return carry

        lax.fori_loop(0, rows_per_tile, body, 0, unroll=False)
        plsc.subcore_barrier()

        _epilogue(s, acc, out_hbm.at[c])

    return agg


def _make_agg(col_split, interpret=False):
    rows_per_tile = RPC // NS if col_split else RPC // (NC * NS)
    return functools.partial(
        pl.kernel,
        mesh=_MESH,
        out_type=jax.ShapeDtypeStruct((2, N, HALF), jnp.float32),
        scratch_types=[
            pltpu.VMEM((rows_per_tile, CH), jnp.int32),
            pltpu.VMEM((rows_per_tile, CH), jnp.int32),
            pltpu.VMEM((CH, HALF), jnp.float32),
            pltpu.VMEM_SHARED((NACC, HALF), jnp.float32),
            pltpu.SemaphoreType.DMA,
        ],
        interpret=interpret,
    )(_make_agg_body(col_split))


_agg_l1 = _make_agg(col_split=True)
_agg_l2 = _make_agg(col_split=False)


_NB = 1000


def _norms(degs_blk):
    outd = degs_blk[0, :, 0:1] + degs_blk[2, :, 0:1]
    ind = degs_blk[1, :, 0:1] + degs_blk[3, :, 0:1]
    ns = jnp.where(outd > 0, lax.rsqrt(jnp.maximum(outd, 1e-12)), 0.0)
    nd = jnp.where(ind > 0, lax.rsqrt(jnp.maximum(ind, 1e-12)), 0.0)
    return ns, nd


def _tc_pre_body(x_ref, w1_ref, degs_ref, out_ref):
    ns, _ = _norms(degs_ref[...])
    y = jnp.dot(x_ref[...], w1_ref[...], preferred_element_type=jnp.float32)
    y = y * ns
    out_ref[0] = y[:, :HALF]
    out_ref[1] = y[:, HALF:]


def _tc_mid_body(agg_ref, degs_ref, b1_ref, w2_ref, out_ref):
    ns, nd = _norms(degs_ref[...])
    h = jnp.concatenate([agg_ref[0], agg_ref[1]], axis=-1)
    h = jnp.maximum(h * nd + b1_ref[...], 0.0)
    y = jnp.dot(h, w2_ref[...], preferred_element_type=jnp.float32)
    out_ref[...] = y * ns


def _tc_post_body(parts_ref, degs_ref, b2_ref, out_ref):
    _, nd = _norms(degs_ref[...])
    out_ref[...] = (parts_ref[0] + parts_ref[1]) * nd + b2_ref[...]


def _tc_pre(x, w1, degs):
    return pl.pallas_call(
        _tc_pre_body,
        grid=(N // _NB,),
        in_specs=[
            pl.BlockSpec((_NB, D_IN), lambda i: (i, 0)),
            pl.BlockSpec((D_IN, D_H), lambda i: (0, 0)),
            pl.BlockSpec((4, _NB, HALF), lambda i: (0, i, 0)),
        ],
        out_specs=pl.BlockSpec((2, _NB, HALF), lambda i: (0, i, 0)),
        out_shape=jax.ShapeDtypeStruct((2, N, HALF), jnp.float32),
    )(x, w1, degs)


def _tc_mid(agg1, degs, b1, w2):
    return pl.pallas_call(
        _tc_mid_body,
        grid=(N // _NB,),
        in_specs=[
            pl.BlockSpec((2, _NB, HALF), lambda i: (0, i, 0)),
            pl.BlockSpec((4, _NB, HALF), lambda i: (0, i, 0)),
            pl.BlockSpec((1, D_H), lambda i: (0, 0)),
            pl.BlockSpec((D_H, D_OUT), lambda i: (0, 0)),
        ],
        out_specs=pl.BlockSpec((_NB, D_OUT), lambda i: (i, 0)),
        out_shape=jax.ShapeDtypeStruct((N, D_OUT), jnp.float32),
    )(agg1, degs, b1, w2)


def _tc_post(parts, degs, b2):
    return pl.pallas_call(
        _tc_post_body,
        grid=(N // _NB,),
        in_specs=[
            pl.BlockSpec((2, _NB, HALF), lambda i: (0, i, 0)),
            pl.BlockSpec((4, _NB, HALF), lambda i: (0, i, 0)),
            pl.BlockSpec((1, D_OUT), lambda i: (0, 0)),
        ],
        out_specs=pl.BlockSpec((_NB, D_OUT), lambda i: (i, 0)),
        out_shape=jax.ShapeDtypeStruct((N, D_OUT), jnp.float32),
    )(parts, degs, b2)


def kernel(features, edge_index, W1, b1, W2, b2):
    src = edge_index[0]
    dst = edge_index[1]
    npad = EP - E
    pad_dummy = N + (jnp.arange(npad, dtype=jnp.int32) % NPAD)
    src_p = jnp.concatenate(
        [src.astype(jnp.int32), jnp.zeros((npad,), jnp.int32)]).reshape(RPC, CH)
    srcd_p = jnp.concatenate([src.astype(jnp.int32), pad_dummy]).reshape(RPC, CH)
    dst_p = jnp.concatenate([dst.astype(jnp.int32), pad_dummy]).reshape(RPC, CH)

    zrows = jnp.zeros((ZROWS, HALF), jnp.float32)
    ones_rows = jnp.ones((CH, HALF), jnp.float32)

    degs = _degrees(srcd_p, dst_p, ones_rows, zrows)
    hw1 = _tc_pre(features, W1, degs)
    agg1 = _agg_l1(src_p, dst_p, hw1, zrows)
    hw2 = _tc_mid(agg1, degs, b1.reshape(1, D_H), W2)
    parts = _agg_l2(src_p, dst_p, hw2, zrows)
    out = _tc_post(parts, degs, b2.reshape(1, D_OUT))
    return out


# speedup vs baseline: 3.3463x; 1.1293x over previous
import functools

import jax
import jax.numpy as jnp
from jax import lax
from jax.experimental import pallas as pl
from jax.experimental.pallas import tpu as pltpu
from jax.experimental.pallas import tpu_sc as plsc

N = 10000
E = 160000
D_IN = 256
D_H = 256
D_OUT = 128
HALF = 128

NC = 2
NS = 16
CH = 128
EP = 163840
RPC = EP // CH
L1_N = RPC // NS
L2_N = RPC // (NC * NS)
DEG_N = RPC // NS
NPAD = 16
ZROWS = 640
NACC = NS * ZROWS
OR_LO = 624
OR_HI = N - 15 * OR_LO

_MESH = plsc.VectorSubcoreMesh(core_axis_name="c", subcore_axis_name="s",
                               num_cores=NC, num_subcores=NS)


def _epilogue(s, acc, out):
    @pl.when(s < 15)
    def _lo():
        r0 = pl.multiple_of(s * OR_LO, 8)
        pltpu.sync_copy(acc.at[pl.ds(r0, OR_LO)], out.at[pl.ds(r0, OR_LO)])

    @pl.when(s == 15)
    def _hi():
        r0 = 15 * OR_LO
        pltpu.sync_copy(acc.at[pl.ds(r0, OR_HI)], out.at[pl.ds(r0, OR_HI)])


def _degrees_body(src_hbm, dst_hbm, ones_hbm, z_hbm, out_hbm,
                  idx, ones_v, acc, sem):
    c = lax.axis_index("c")
    s = lax.axis_index("s")

    z0 = pl.multiple_of(s * ZROWS, 8)
    pltpu.sync_copy(z_hbm, acc.at[pl.ds(z0, ZROWS)])
    base = pl.multiple_of(s * DEG_N, 8)

    @pl.when(c == 0)
    def _ld0():
        pltpu.sync_copy(src_hbm.at[pl.ds(base, DEG_N)], idx)

    @pl.when(c == 1)
    def _ld1():
        pltpu.sync_copy(dst_hbm.at[pl.ds(base, DEG_N)], idx)

    pltpu.sync_copy(ones_hbm, ones_v)
    plsc.subcore_barrier()

    def outer(o, carry):
        for k in range(8):
            pltpu.async_copy(ones_v, acc.at[idx.at[o * 8 + k]], sem,
                             add=True)
        for k in range(8):
            pltpu.make_async_copy(ones_v, acc.at[idx.at[o * 8 + k]],
                                  sem).wait()
        return carry

    lax.fori_loop(0, DEG_N // 8, outer, 0, unroll=False)
    plsc.subcore_barrier()
    _epilogue(s, acc, out_hbm.at[c])


_degrees = functools.partial(
    pl.kernel,
    mesh=_MESH,
    out_type=jax.ShapeDtypeStruct((2, N, HALF), jnp.float32),
    scratch_types=[
        pltpu.VMEM((DEG_N, CH), jnp.int32),
        pltpu.VMEM((CH, HALF), jnp.float32),
        pltpu.VMEM_SHARED((NACC, HALF), jnp.float32),
        pltpu.SemaphoreType.DMA,
    ],
)(_degrees_body)


PH = 40


def _make_agg_body(col_split):
    phases = (L1_N if col_split else L2_N) // PH

    def agg(src_hbm, dst_hbm, tbl_hbm, z_hbm, out_hbm,
            sidx, didx, rowbuf, acc, gsem, ssem):
        c = lax.axis_index("c")
        s = lax.axis_index("s")

        z0 = pl.multiple_of(s * ZROWS, 8)
        pltpu.sync_copy(z_hbm, acc.at[pl.ds(z0, ZROWS)])
        if col_split:
            base = s * L1_N
        else:
            base = c * (RPC // NC) + s * L2_N
        plsc.subcore_barrier()

        def tbl():
            return tbl_hbm.at[c] if col_split else tbl_hbm

        for p in range(phases):
            bp = pl.multiple_of(base + p * PH, 8)
            pltpu.sync_copy(src_hbm.at[pl.ds(bp, PH)], sidx)
            pltpu.sync_copy(dst_hbm.at[pl.ds(bp, PH)], didx)

            pltpu.async_copy(tbl().at[sidx.at[0]], rowbuf.at[0], gsem)

            def body(j, carry):
                b = lax.rem(j, 2)
                pltpu.make_async_copy(tbl().at[sidx.at[j]], rowbuf.at[b],
                                      gsem).wait()

                @pl.when(j >= 1)
                def _drain_prev():
                    pltpu.make_async_copy(rowbuf.at[1 - b],
                                          acc.at[didx.at[j]], ssem).wait()

                @pl.when(j + 1 < PH)
                def _fire_next_gather():
                    pltpu.async_copy(tbl().at[sidx.at[j + 1]],
                                     rowbuf.at[1 - b], gsem)

                pltpu.async_copy(rowbuf.at[b], acc.at[didx.at[j]], ssem,
                                 add=True)
                return carry

            lax.fori_loop(0, PH, body, 0, unroll=False)
            pltpu.make_async_copy(rowbuf.at[(PH - 1) % 2],
                                  acc.at[didx.at[PH - 1]], ssem).wait()

        plsc.subcore_barrier()
        _epilogue(s, acc, out_hbm.at[c])

    return agg


def _make_agg(col_split):
    return functools.partial(
        pl.kernel,
        mesh=_MESH,
        out_type=jax.ShapeDtypeStruct((2, N, HALF), jnp.float32),
        scratch_types=[
            pltpu.VMEM((PH, CH), jnp.int32),
            pltpu.VMEM((PH, CH), jnp.int32),
            pltpu.VMEM((2, CH, HALF), jnp.float32),
            pltpu.VMEM_SHARED((NACC, HALF), jnp.float32),
            pltpu.SemaphoreType.DMA,
            pltpu.SemaphoreType.DMA,
        ],
    )(_make_agg_body(col_split))


_agg_l1 = _make_agg(col_split=True)
_agg_l2 = _make_agg(col_split=False)


_NB = 1000


def _norms(degs_blk):
    outd = degs_blk[0, :, 0:1]
    ind = degs_blk[1, :, 0:1]
    ns = jnp.where(outd > 0, lax.rsqrt(jnp.maximum(outd, 1e-12)), 0.0)
    nd = jnp.where(ind > 0, lax.rsqrt(jnp.maximum(ind, 1e-12)), 0.0)
    return ns, nd


def _tc_pre_body(x_ref, w1_ref, degs_ref, out_ref):
    ns, _ = _norms(degs_ref[...])
    y = jnp.dot(x_ref[...], w1_ref[...], preferred_element_type=jnp.float32)
    y = y * ns
    out_ref[0] = y[:, :HALF]
    out_ref[1] = y[:, HALF:]


def _tc_mid_body(agg_ref, degs_ref, b1_ref, w2_ref, out_ref):
    ns, nd = _norms(degs_ref[...])
    h = jnp.concatenate([agg_ref[0], agg_ref[1]], axis=-1)
    h = jnp.maximum(h * nd + b1_ref[...], 0.0)
    y = jnp.dot(h, w2_ref[...], preferred_element_type=jnp.float32)
    out_ref[...] = y * ns


def _tc_post_body(parts_ref, degs_ref, b2_ref, out_ref):
    _, nd = _norms(degs_ref[...])
    out_ref[...] = (parts_ref[0] + parts_ref[1]) * nd + b2_ref[...]


def _tc_pre(x, w1, degs):
    return pl.pallas_call(
        _tc_pre_body,
        grid=(N // _NB,),
        in_specs=[
            pl.BlockSpec((_NB, D_IN), lambda i: (i, 0)),
            pl.BlockSpec((D_IN, D_H), lambda i: (0, 0)),
            pl.BlockSpec((2, _NB, HALF), lambda i: (0, i, 0)),
        ],
        out_specs=pl.BlockSpec((2, _NB, HALF), lambda i: (0, i, 0)),
        out_shape=jax.ShapeDtypeStruct((2, N, HALF), jnp.float32),
    )(x, w1, degs)


def _tc_mid(agg1, degs, b1, w2):
    return pl.pallas_call(
        _tc_mid_body,
        grid=(N // _NB,),
        in_specs=[
            pl.BlockSpec((2, _NB, HALF), lambda i: (0, i, 0)),
            pl.BlockSpec((2, _NB, HALF), lambda i: (0, i, 0)),
            pl.BlockSpec((1, D_H), lambda i: (0, 0)),
            pl.BlockSpec((D_H, D_OUT), lambda i: (0, 0)),
        ],
        out_specs=pl.BlockSpec((_NB, D_OUT), lambda i: (i, 0)),
        out_shape=jax.ShapeDtypeStruct((N, D_OUT), jnp.float32),
    )(agg1, degs, b1, w2)


def _tc_post(parts, degs, b2):
    return pl.pallas_call(
        _tc_post_body,
        grid=(N // _NB,),
        in_specs=[
            pl.BlockSpec((2, _NB, HALF), lambda i: (0, i, 0)),
            pl.BlockSpec((2, _NB, HALF), lambda i: (0, i, 0)),
            pl.BlockSpec((1, D_OUT), lambda i: (0, 0)),
        ],
        out_specs=pl.BlockSpec((_NB, D_OUT), lambda i: (i, 0)),
        out_shape=jax.ShapeDtypeStruct((N, D_OUT), jnp.float32),
    )(parts, degs, b2)


def kernel(features, edge_index, W1, b1, W2, b2):
    src = edge_index[0]
    dst = edge_index[1]
    npad = EP - E
    pad_dummy = N + (jnp.arange(npad, dtype=jnp.int32) % NPAD)
    src_p = jnp.concatenate(
        [src.astype(jnp.int32), jnp.zeros((npad,), jnp.int32)]).reshape(RPC, CH)
    srcd_p = jnp.concatenate([src.astype(jnp.int32), pad_dummy]).reshape(RPC, CH)
    dst_p = jnp.concatenate([dst.astype(jnp.int32), pad_dummy]).reshape(RPC, CH)

    zrows = jnp.zeros((ZROWS, HALF), jnp.float32)
    ones_rows = jnp.ones((CH, HALF), jnp.float32)

    degs = _degrees(srcd_p, dst_p, ones_rows, zrows)
    hw1 = _tc_pre(features, W1, degs)
    agg1 = _agg_l1(src_p, dst_p, hw1, zrows)
    hw2 = _tc_mid(agg1, degs, b1.reshape(1, D_H), W2)
    parts = _agg_l2(src_p, dst_p, hw2, zrows)
    out = _tc_post(parts, degs, b2.reshape(1, D_OUT))
    return out


# speedup vs baseline: 6.9895x; 2.0887x over previous
import functools

import jax
import jax.numpy as jnp
from jax import lax
from jax.experimental import pallas as pl
from jax.experimental.pallas import tpu as pltpu
from jax.experimental.pallas import tpu_sc as plsc

N = 10000
E = 160000
D_IN = 256
D_H = 256
D_OUT = 128
HALF = 128

NC = 2
NS = 16
CH = 128
EP = 163840
RPC = EP // CH
L1_N = RPC // NS
L2_N = RPC // (NC * NS)
DEG_N = RPC // NS
NPAD = 16
ZROWS = 640
NACC = NS * ZROWS
OR_LO = 624
OR_HI = N - 15 * OR_LO

_MESH = plsc.VectorSubcoreMesh(core_axis_name="c", subcore_axis_name="s",
                               num_cores=NC, num_subcores=NS)


def _epilogue(s, acc, out):
    @pl.when(s < 15)
    def _lo():
        r0 = pl.multiple_of(s * OR_LO, 8)
        pltpu.sync_copy(acc.at[pl.ds(r0, OR_LO)], out.at[pl.ds(r0, OR_LO)])

    @pl.when(s == 15)
    def _hi():
        r0 = 15 * OR_LO
        pltpu.sync_copy(acc.at[pl.ds(r0, OR_HI)], out.at[pl.ds(r0, OR_HI)])


def _degrees_body(src_hbm, dst_hbm, ones_hbm, z_hbm, out_hbm,
                  idx, ones_v, acc, sem):
    c = lax.axis_index("c")
    s = lax.axis_index("s")

    z0 = pl.multiple_of(s * ZROWS, 8)
    pltpu.sync_copy(z_hbm, acc.at[pl.ds(z0, ZROWS)])
    base = pl.multiple_of(s * DEG_N, 8)

    @pl.when(c == 0)
    def _ld0():
        pltpu.sync_copy(src_hbm.at[pl.ds(base, DEG_N)], idx)

    @pl.when(c == 1)
    def _ld1():
        pltpu.sync_copy(dst_hbm.at[pl.ds(base, DEG_N)], idx)

    pltpu.sync_copy(ones_hbm, ones_v)
    plsc.subcore_barrier()

    def outer(o, carry):
        for k in range(8):
            pltpu.async_copy(ones_v, acc.at[idx.at[o * 8 + k]], sem,
                             add=True)
        for k in range(8):
            pltpu.make_async_copy(ones_v, acc.at[idx.at[o * 8 + k]],
                                  sem).wait()
        return carry

    lax.fori_loop(0, DEG_N // 8, outer, 0, unroll=False)
    plsc.subcore_barrier()
    _epilogue(s, acc, out_hbm.at[c])


_degrees = functools.partial(
    pl.kernel,
    mesh=_MESH,
    out_type=jax.ShapeDtypeStruct((2, N, HALF), jnp.float32),
    scratch_types=[
        pltpu.VMEM((DEG_N, CH), jnp.int32),
        pltpu.VMEM((CH, HALF), jnp.float32),
        pltpu.VMEM_SHARED((NACC, HALF), jnp.float32),
        pltpu.SemaphoreType.DMA,
    ],
)(_degrees_body)


PH = 40


def _make_agg_body(col_split):
    phases = (L1_N if col_split else L2_N) // PH

    def agg(src_hbm, dst_hbm, tbl_hbm, z_hbm, out_hbm,
            sidx, didx, rowbuf, acc, gsem, ssem):
        c = lax.axis_index("c")
        s = lax.axis_index("s")

        z0 = pl.multiple_of(s * ZROWS, 8)
        pltpu.sync_copy(z_hbm, acc.at[pl.ds(z0, ZROWS)])
        if col_split:
            base = s * L1_N
        else:
            base = c * (RPC // NC) + s * L2_N
        plsc.subcore_barrier()

        def tbl():
            return tbl_hbm.at[c] if col_split else tbl_hbm

        for p in range(phases):
            bp = pl.multiple_of(base + p * PH, 8)
            pltpu.sync_copy(src_hbm.at[pl.ds(bp, PH)], sidx)
            pltpu.sync_copy(dst_hbm.at[pl.ds(bp, PH)], didx)

            pltpu.async_copy(tbl().at[sidx.at[0]], rowbuf.at[0], gsem)

            def body(j, carry):
                b = lax.rem(j, 2)
                pltpu.make_async_copy(tbl().at[sidx.at[j]], rowbuf.at[b],
                                      gsem).wait()

                @pl.when(j >= 1)
                def _drain_prev():
                    pltpu.make_async_copy(rowbuf.at[1 - b],
                                          acc.at[didx.at[j]], ssem).wait()

                @pl.when(j + 1 < PH)
                def _fire_next_gather():
                    pltpu.async_copy(tbl().at[sidx.at[j + 1]],
                                     rowbuf.at[1 - b], gsem)

                pltpu.async_copy(rowbuf.at[b], acc.at[didx.at[j]], ssem,
                                 add=True)
                return carry

            lax.fori_loop(0, PH, body, 0, unroll=False)
            pltpu.make_async_copy(rowbuf.at[(PH - 1) % 2],
                                  acc.at[didx.at[PH - 1]], ssem).wait()

        plsc.subcore_barrier()
        _epilogue(s, acc, out_hbm.at[c])

    return agg


def _make_agg(col_split):
    return functools.partial(
        pl.kernel,
        mesh=_MESH,
        out_type=jax.ShapeDtypeStruct((2, N, HALF), jnp.float32),
        scratch_types=[
            pltpu.VMEM((PH, CH), jnp.int32),
            pltpu.VMEM((PH, CH), jnp.int32),
            pltpu.VMEM((2, CH, HALF), jnp.float32),
            pltpu.VMEM_SHARED((NACC, HALF), jnp.float32),
            pltpu.SemaphoreType.DMA,
            pltpu.SemaphoreType.DMA,
        ],
    )(_make_agg_body(col_split))


_agg_l1 = _make_agg(col_split=True)
_agg_l2 = _make_agg(col_split=False)


_NB = 1000


def _norms(degs_blk):
    outd = degs_blk[0, :, 0:1]
    ind = degs_blk[1, :, 0:1]
    ns = jnp.where(outd > 0, lax.rsqrt(jnp.maximum(outd, 1e-12)), 0.0)
    nd = jnp.where(ind > 0, lax.rsqrt(jnp.maximum(ind, 1e-12)), 0.0)
    return ns, nd


def _tc_pre_body(x_ref, w1_ref, degs_ref, out_ref):
    ns, _ = _norms(degs_ref[...])
    y = jnp.dot(x_ref[...], w1_ref[...], preferred_element_type=jnp.float32)
    y = y * ns
    out_ref[0] = y[:, :HALF]
    out_ref[1] = y[:, HALF:]


def _tc_mid_body(agg_ref, degs_ref, b1_ref, w2_ref, out_ref):
    ns, nd = _norms(degs_ref[...])
    h = jnp.concatenate([agg_ref[0], agg_ref[1]], axis=-1)
    h = jnp.maximum(h * nd + b1_ref[...], 0.0)
    y = jnp.dot(h, w2_ref[...], preferred_element_type=jnp.float32)
    out_ref[...] = y * ns


def _tc_post_body(parts_ref, degs_ref, b2_ref, out_ref):
    _, nd = _norms(degs_ref[...])
    out_ref[...] = (parts_ref[0] + parts_ref[1]) * nd + b2_ref[...]


def _tc_pre(x, w1, degs):
    return pl.pallas_call(
        _tc_pre_body,
        grid=(N // _NB,),
        in_specs=[
            pl.BlockSpec((_NB, D_IN), lambda i: (i, 0)),
            pl.BlockSpec((D_IN, D_H), lambda i: (0, 0)),
            pl.BlockSpec((2, _NB, HALF), lambda i: (0, i, 0)),
        ],
        out_specs=pl.BlockSpec((2, _NB, HALF), lambda i: (0, i, 0)),
        out_shape=jax.ShapeDtypeStruct((2, N, HALF), jnp.float32),
    )(x, w1, degs)


def _tc_mid(agg1, degs, b1, w2):
    return pl.pallas_call(
        _tc_mid_body,
        grid=(N // _NB,),
        in_specs=[
            pl.BlockSpec((2, _NB, HALF), lambda i: (0, i, 0)),
            pl.BlockSpec((2, _NB, HALF), lambda i: (0, i, 0)),
            pl.BlockSpec((1, D_H), lambda i: (0, 0)),
            pl.BlockSpec((D_H, D_OUT), lambda i: (0, 0)),
        ],
        out_specs=pl.BlockSpec((_NB, D_OUT), lambda i: (i, 0)),
        out_shape=jax.ShapeDtypeStruct((N, D_OUT), jnp.float32),
    )(agg1, degs, b1, w2)


def _tc_post(parts, degs, b2):
    return pl.pallas_call(
        _tc_post_body,
        grid=(N // _NB,),
        in_specs=[
            pl.BlockSpec((2, _NB, HALF), lambda i: (0, i, 0)),
            pl.BlockSpec((2, _NB, HALF), lambda i: (0, i, 0)),
            pl.BlockSpec((1, D_OUT), lambda i: (0, 0)),
        ],
        out_specs=pl.BlockSpec((_NB, D_OUT), lambda i: (i, 0)),
        out_shape=jax.ShapeDtypeStruct((N, D_OUT), jnp.float32),
    )(parts, degs, b2)


def kernel(features, edge_index, W1, b1, W2, b2):
    src = edge_index[0]
    dst = edge_index[1]
    npad = EP - E
    pad_dummy = N + (jnp.arange(npad, dtype=jnp.int32) % NPAD)
    pad_spread = jnp.arange(npad, dtype=jnp.int32) % N
    src_p = jnp.concatenate(
        [src.astype(jnp.int32), pad_spread]).reshape(RPC, CH)
    srcd_p = jnp.concatenate([src.astype(jnp.int32), pad_dummy]).reshape(RPC, CH)
    dst_p = jnp.concatenate([dst.astype(jnp.int32), pad_dummy]).reshape(RPC, CH)

    zrows = jnp.zeros((ZROWS, HALF), jnp.float32)
    ones_rows = jnp.ones((CH, HALF), jnp.float32)

    degs = _degrees(srcd_p, dst_p, ones_rows, zrows)
    hw1 = _tc_pre(features, W1, degs)
    agg1 = _agg_l1(src_p, dst_p, hw1, zrows)
    hw2 = _tc_mid(agg1, degs, b1.reshape(1, D_H), W2)
    parts = _agg_l2(src_p, dst_p, hw2, zrows)
    out = _tc_post(parts, degs, b2.reshape(1, D_OUT))
    return out


# speedup vs baseline: 8.3413x; 1.1934x over previous
import functools

import jax
import jax.numpy as jnp
from jax import lax
from jax.experimental import pallas as pl
from jax.experimental.pallas import tpu as pltpu
from jax.experimental.pallas import tpu_sc as plsc

N = 10000
E = 160000
D_IN = 256
D_H = 256
D_OUT = 128
HALF = 128

NC = 2
NS = 16
CH = 128
EP = 163840
RPC = EP // CH
L1_N = RPC // NS
L2_N = RPC // (NC * NS)
DEG_N = RPC // NS
NPAD = 16
ZROWS = 640
NACC = NS * ZROWS
OR_LO = 624
OR_HI = N - 15 * OR_LO

_MESH = plsc.VectorSubcoreMesh(core_axis_name="c", subcore_axis_name="s",
                               num_cores=NC, num_subcores=NS)


def _epilogue(s, acc, out):
    @pl.when(s < 15)
    def _lo():
        r0 = pl.multiple_of(s * OR_LO, 8)
        pltpu.sync_copy(acc.at[pl.ds(r0, OR_LO)], out.at[pl.ds(r0, OR_LO)])

    @pl.when(s == 15)
    def _hi():
        r0 = 15 * OR_LO
        pltpu.sync_copy(acc.at[pl.ds(r0, OR_HI)], out.at[pl.ds(r0, OR_HI)])


HR = NACC // 128


def _degrees_body(src_hbm, dst_hbm, z_hbm, out_hbm,
                  idx, hist, rowid, acc):
    c = lax.axis_index("c")
    s = lax.axis_index("s")

    base = pl.multiple_of(s * DEG_N, 8)

    @pl.when(c == 0)
    def _ld0():
        pltpu.sync_copy(src_hbm.at[pl.ds(base, DEG_N)], idx)

    @pl.when(c == 1)
    def _ld1():
        pltpu.sync_copy(dst_hbm.at[pl.ds(base, DEG_N)], idx)

    pltpu.sync_copy(z_hbm.at[pl.ds(0, HR)], hist)

    @pl.when(s == 0)
    def _zero_acc():
        pltpu.sync_copy(z_hbm.at[pl.ds(0, HR)], acc)

    iota16 = lax.iota(jnp.int32, 16)
    for k in range(HR // 16):
        rowid[pl.ds(k * 16, 16)] = iota16 + (k * 16)

    ones = jnp.ones((16,), jnp.float32)

    def body(j, carry):
        def inner(k, carry2):
            iv = idx[j, pl.ds(pl.multiple_of(k * 16, 16), 16)]
            plsc.addupdate_scatter(
                hist, [lax.shift_right_logical(iv, 7),
                       lax.bitwise_and(iv, 127)], ones)
            return carry2

        lax.fori_loop(0, CH // 16, inner, 0, unroll=False)
        return carry

    lax.fori_loop(0, DEG_N, body, 0, unroll=False)
    plsc.subcore_barrier()
    pltpu.sync_copy(hist, acc.at[rowid], add=True)
    plsc.subcore_barrier()

    @pl.when(s == 0)
    def _wr():
        pltpu.sync_copy(acc, out_hbm.at[c])


_degrees = functools.partial(
    pl.kernel,
    mesh=_MESH,
    out_type=jax.ShapeDtypeStruct((2, HR, 128), jnp.float32),
    scratch_types=[
        pltpu.VMEM((DEG_N, CH), jnp.int32),
        pltpu.VMEM((HR, 128), jnp.float32),
        pltpu.VMEM((HR,), jnp.int32),
        pltpu.VMEM_SHARED((HR, 128), jnp.float32),
    ],
    compiler_params=pltpu.CompilerParams(needs_layout_passes=False),
)(_degrees_body)


PH = 40


def _make_agg_body(col_split):
    phases = (L1_N if col_split else L2_N) // PH

    def agg(src_hbm, dst_hbm, tbl_hbm, z_hbm, out_hbm,
            sidx, didx, rowbuf, acc, gsem, ssem):
        c = lax.axis_index("c")
        s = lax.axis_index("s")

        z0 = pl.multiple_of(s * ZROWS, 8)
        pltpu.sync_copy(z_hbm, acc.at[pl.ds(z0, ZROWS)])
        if col_split:
            base = s * L1_N
        else:
            base = c * (RPC // NC) + s * L2_N
        plsc.subcore_barrier()

        def tbl():
            return tbl_hbm.at[c] if col_split else tbl_hbm

        for p in range(phases):
            bp = pl.multiple_of(base + p * PH, 8)
            pltpu.sync_copy(src_hbm.at[pl.ds(bp, PH)], sidx)
            pltpu.sync_copy(dst_hbm.at[pl.ds(bp, PH)], didx)

            pltpu.async_copy(tbl().at[sidx.at[0]], rowbuf.at[0], gsem)

            def body(j, carry):
                b = lax.rem(j, 2)
                pltpu.make_async_copy(tbl().at[sidx.at[j]], rowbuf.at[b],
                                      gsem).wait()

                @pl.when(j >= 1)
                def _drain_prev():
                    pltpu.make_async_copy(rowbuf.at[1 - b],
                                          acc.at[didx.at[j]], ssem).wait()

                @pl.when(j + 1 < PH)
                def _fire_next_gather():
                    pltpu.async_copy(tbl().at[sidx.at[j + 1]],
                                     rowbuf.at[1 - b], gsem)

                pltpu.async_copy(rowbuf.at[b], acc.at[didx.at[j]], ssem,
                                 add=True)
                return carry

            lax.fori_loop(0, PH, body, 0, unroll=False)
            pltpu.make_async_copy(rowbuf.at[(PH - 1) % 2],
                                  acc.at[didx.at[PH - 1]], ssem).wait()

        plsc.subcore_barrier()
        _epilogue(s, acc, out_hbm.at[c])

    return agg


def _make_agg(col_split):
    return functools.partial(
        pl.kernel,
        mesh=_MESH,
        out_type=jax.ShapeDtypeStruct((2, N, HALF), jnp.float32),
        scratch_types=[
            pltpu.VMEM((PH, CH), jnp.int32),
            pltpu.VMEM((PH, CH), jnp.int32),
            pltpu.VMEM((2, CH, HALF), jnp.float32),
            pltpu.VMEM_SHARED((NACC, HALF), jnp.float32),
            pltpu.SemaphoreType.DMA,
            pltpu.SemaphoreType.DMA,
        ],
    )(_make_agg_body(col_split))


_agg_l1 = _make_agg(col_split=True)
_agg_l2 = _make_agg(col_split=False)


_NB = 1000


_HB = HR
_NPB = _HB * 128


def _normprep_body(degs_ref, out_ref):
    lane = lax.broadcasted_iota(jnp.int32, (_NPB, 128), 1)
    sub = lax.broadcasted_iota(jnp.int32, (_NPB, 128), 0)
    diag = lane == lax.rem(sub, 128)
    for a in range(2):
        d = degs_ref[a]
        nrm = jnp.where(d > 0, lax.rsqrt(jnp.maximum(d, 1e-12)), 0.0)
        rep = jnp.broadcast_to(nrm[:, None, :], (_HB, 128, 128))
        rep = rep.reshape(_NPB, 128)
        out_ref[a] = jnp.sum(jnp.where(diag, rep, 0.0), axis=1,
                             keepdims=True)


def _normprep(degs):
    return pl.pallas_call(
        _normprep_body,
        grid=(HR // _HB,),
        in_specs=[pl.BlockSpec((2, _HB, 128), lambda i: (0, 0, 0))],
        out_specs=pl.BlockSpec((2, _NPB, 1), lambda i: (0, 0, 0)),
        out_shape=jax.ShapeDtypeStruct((2, NACC, 1), jnp.float32),
    )(degs)


def _tc_pre_body(x_ref, w1_ref, nrm_ref, out_ref):
    ns = nrm_ref[0]
    y = jnp.dot(x_ref[...], w1_ref[...], preferred_element_type=jnp.float32)
    y = y * ns
    out_ref[0] = y[:, :HALF]
    out_ref[1] = y[:, HALF:]


def _tc_mid_body(agg_ref, nrm_ref, b1_ref, w2_ref, out_ref):
    ns = nrm_ref[0]
    nd = nrm_ref[1]
    h = jnp.concatenate([agg_ref[0], agg_ref[1]], axis=-1)
    h = jnp.maximum(h * nd + b1_ref[...], 0.0)
    y = jnp.dot(h, w2_ref[...], preferred_element_type=jnp.float32)
    out_ref[...] = y * ns


def _tc_post_body(parts_ref, nrm_ref, b2_ref, out_ref):
    nd = nrm_ref[1]
    out_ref[...] = (parts_ref[0] + parts_ref[1]) * nd + b2_ref[...]


def _tc_pre(x, w1, nrm):
    return pl.pallas_call(
        _tc_pre_body,
        grid=(N // _NB,),
        in_specs=[
            pl.BlockSpec((_NB, D_IN), lambda i: (i, 0)),
            pl.BlockSpec((D_IN, D_H), lambda i: (0, 0)),
            pl.BlockSpec((2, _NB, 1), lambda i: (0, i, 0)),
        ],
        out_specs=pl.BlockSpec((2, _NB, HALF), lambda i: (0, i, 0)),
        out_shape=jax.ShapeDtypeStruct((2, N, HALF), jnp.float32),
    )(x, w1, nrm)


def _tc_mid(agg1, degs, b1, w2):
    return pl.pallas_call(
        _tc_mid_body,
        grid=(N // _NB,),
        in_specs=[
            pl.BlockSpec((2, _NB, HALF), lambda i: (0, i, 0)),
            pl.BlockSpec((2, _NB, 1), lambda i: (0, i, 0)),
            pl.BlockSpec((1, D_H), lambda i: (0, 0)),
            pl.BlockSpec((D_H, D_OUT), lambda i: (0, 0)),
        ],
        out_specs=pl.BlockSpec((_NB, D_OUT), lambda i: (i, 0)),
        out_shape=jax.ShapeDtypeStruct((N, D_OUT), jnp.float32),
    )(agg1, degs, b1, w2)


def _tc_post(parts, degs, b2):
    return pl.pallas_call(
        _tc_post_body,
        grid=(N // _NB,),
        in_specs=[
            pl.BlockSpec((2, _NB, HALF), lambda i: (0, i, 0)),
            pl.BlockSpec((2, _NB, 1), lambda i: (0, i, 0)),
            pl.BlockSpec((1, D_OUT), lambda i: (0, 0)),
        ],
        out_specs=pl.BlockSpec((_NB, D_OUT), lambda i: (i, 0)),
        out_shape=jax.ShapeDtypeStruct((N, D_OUT), jnp.float32),
    )(parts, degs, b2)


def kernel(features, edge_index, W1, b1, W2, b2):
    src = edge_index[0]
    dst = edge_index[1]
    npad = EP - E
    pad_dummy = N + (jnp.arange(npad, dtype=jnp.int32) % NPAD)
    pad_spread = jnp.arange(npad, dtype=jnp.int32) % N
    src_p = jnp.concatenate(
        [src.astype(jnp.int32), pad_spread]).reshape(RPC, CH)
    srcd_p = jnp.concatenate([src.astype(jnp.int32), pad_dummy]).reshape(RPC, CH)
    dst_p = jnp.concatenate([dst.astype(jnp.int32), pad_dummy]).reshape(RPC, CH)

    zrows = jnp.zeros((ZROWS, HALF), jnp.float32)

    degs = _degrees(srcd_p, dst_p, zrows)
    nrm = _normprep(degs)
    hw1 = _tc_pre(features, W1, nrm)
    agg1 = _agg_l1(src_p, dst_p, hw1, zrows)
    hw2 = _tc_mid(agg1, nrm, b1.reshape(1, D_H), W2)
    parts = _agg_l2(src_p, dst_p, hw2, zrows)
    out = _tc_post(parts, nrm, b2.reshape(1, D_OUT))
    return out
